# speedup vs baseline: 3.1020x; 3.1020x over previous
import jax
import jax.numpy as jnp
from jax.experimental import pallas as pl
from jax.experimental.pallas import tpu as pltpu
from jax.experimental.pallas import tpu_sc as plsc

EMBED_DIM = 128
WINDOW = 128


def kernel(x, table):
    batch, seq = x.shape
    num_indices = batch * seq
    idx = x.reshape(1, num_indices).astype(jnp.int32)

    mesh = plsc.VectorSubcoreMesh(core_axis_name="core", subcore_axis_name="subcore")

    @pl.kernel(
        out_type=jax.ShapeDtypeStruct((num_indices, EMBED_DIM), table.dtype),
        mesh=mesh,
    )
    def gather_kernel(table_hbm, idx_hbm, out_hbm):
        def body(idx_vmem, out_vmem):
            pltpu.sync_copy(table_hbm.at[idx_vmem.at[0]], out_vmem)

        pltpu.emit_pipeline(
            body,
            grid=(num_indices // WINDOW,),
            in_specs=[pl.BlockSpec((1, WINDOW), index_map=lambda i: (0, i))],
            out_specs=[pl.BlockSpec((WINDOW, EMBED_DIM), index_map=lambda i: (i, 0))],
            core_axis_name=("core", "subcore"),
            dimension_semantics=(pltpu.PARALLEL,),
        )(idx_hbm, out_hbm)

    out = gather_kernel(table, idx)
    return out.reshape(batch, seq, EMBED_DIM)


# speedup vs baseline: 3.2879x; 1.0599x over previous
import jax
import jax.numpy as jnp
from jax.experimental import pallas as pl
from jax.experimental.pallas import tpu as pltpu
from jax.experimental.pallas import tpu_sc as plsc

EMBED_DIM = 128
WINDOW = 256


def kernel(x, table):
    batch, seq = x.shape
    num_indices = batch * seq
    idx = x.reshape(1, num_indices).astype(jnp.int32)

    mesh = plsc.VectorSubcoreMesh(core_axis_name="core", subcore_axis_name="subcore")

    @pl.kernel(
        out_type=jax.ShapeDtypeStruct((num_indices, EMBED_DIM), table.dtype),
        mesh=mesh,
    )
    def gather_kernel(table_hbm, idx_hbm, out_hbm):
        def body(idx_vmem, out_vmem):
            pltpu.sync_copy(table_hbm.at[idx_vmem.at[0]], out_vmem)

        pltpu.emit_pipeline(
            body,
            grid=(num_indices // WINDOW,),
            in_specs=[pl.BlockSpec((1, WINDOW), index_map=lambda i: (0, i))],
            out_specs=[pl.BlockSpec((WINDOW, EMBED_DIM), index_map=lambda i: (i, 0))],
            core_axis_name=("core", "subcore"),
            dimension_semantics=(pltpu.PARALLEL,),
        )(idx_hbm, out_hbm)

    out = gather_kernel(table, idx)
    return out.reshape(batch, seq, EMBED_DIM)


# speedup vs baseline: 5.5932x; 1.7011x over previous
import jax
import jax.numpy as jnp
from jax import lax
from jax.experimental import pallas as pl
from jax.experimental.pallas import tpu as pltpu
from jax.experimental.pallas import tpu_sc as plsc

EMBED_DIM = 128
NUM_CORES = 2
NUM_SUBCORES = 16
NUM_WORKERS = NUM_CORES * NUM_SUBCORES
ROWS_PER_GATHER = 4


def kernel(x, table):
    batch, seq = x.shape
    num_idx = batch * seq
    idx = x.reshape(1, num_idx).astype(jnp.int32)

    rows_per_worker = batch // NUM_WORKERS
    idx_per_worker = rows_per_worker * seq
    gw = ROWS_PER_GATHER * seq
    n_gathers = rows_per_worker // ROWS_PER_GATHER

    mesh = plsc.VectorSubcoreMesh(core_axis_name="core", subcore_axis_name="subcore")

    @pl.kernel(
        out_type=jax.ShapeDtypeStruct((batch, seq, EMBED_DIM), table.dtype),
        mesh=mesh,
        scratch_types=[
            pltpu.VMEM((idx_per_worker,), jnp.int32),
            pltpu.VMEM((gw, EMBED_DIM), jnp.float32),
            pltpu.VMEM((gw, EMBED_DIM), jnp.float32),
            pltpu.SemaphoreType.DMA,
            pltpu.SemaphoreType.DMA,
            pltpu.SemaphoreType.DMA,
        ],
    )
    def gather_kernel(table_hbm, idx_hbm, out_hbm, idx_v, buf0, buf1, gsem, osem0, osem1):
        wid = lax.axis_index("subcore") * NUM_CORES + lax.axis_index("core")
        row_base = wid * rows_per_worker

        pltpu.sync_copy(idx_hbm.at[0, pl.ds(row_base * seq, idx_per_worker)], idx_v)

        bufs = (buf0, buf1)
        osems = (osem0, osem1)

        def start_gather(g):
            return pltpu.async_copy(
                table_hbm.at[idx_v.at[pl.ds(g * gw, gw)]], bufs[g % 2], gsem
            )

        out_handles = [[], []]
        gather_handle = start_gather(0)
        for g in range(n_gathers):
            gather_handle.wait()
            if g + 1 < n_gathers:
                nxt = (g + 1) % 2
                for h in out_handles[nxt]:
                    h.wait()
                out_handles[nxt] = []
                gather_handle = start_gather(g + 1)
            buf = bufs[g % 2]
            for j in range(ROWS_PER_GATHER):
                out_handles[g % 2].append(
                    pltpu.async_copy(
                        buf.at[pl.ds(j * seq, seq)],
                        out_hbm.at[row_base + g * ROWS_PER_GATHER + j],
                        osems[g % 2],
                    )
                )
        for side in out_handles:
            for h in side:
                h.wait()

    return gather_kernel(table, idx)


# speedup vs baseline: 5.7367x; 1.0257x over previous
import jax
import jax.numpy as jnp
from jax import lax
from jax.experimental import pallas as pl
from jax.experimental.pallas import tpu as pltpu
from jax.experimental.pallas import tpu_sc as plsc

EMBED_DIM = 128
NUM_CORES = 2
NUM_SUBCORES = 16
NUM_WORKERS = NUM_CORES * NUM_SUBCORES
ROWS_PER_GATHER = 8


def kernel(x, table):
    batch, seq = x.shape
    num_idx = batch * seq
    idx = x.reshape(1, num_idx).astype(jnp.int32)

    rows_per_worker = batch // NUM_WORKERS
    idx_per_worker = rows_per_worker * seq
    gw = ROWS_PER_GATHER * seq
    n_gathers = rows_per_worker // ROWS_PER_GATHER

    mesh = plsc.VectorSubcoreMesh(core_axis_name="core", subcore_axis_name="subcore")

    @pl.kernel(
        out_type=jax.ShapeDtypeStruct((batch, seq, EMBED_DIM), table.dtype),
        mesh=mesh,
        scratch_types=[
            pltpu.VMEM((idx_per_worker,), jnp.int32),
            pltpu.VMEM((gw, EMBED_DIM), jnp.float32),
            pltpu.VMEM((gw, EMBED_DIM), jnp.float32),
            pltpu.SemaphoreType.DMA,
            pltpu.SemaphoreType.DMA,
            pltpu.SemaphoreType.DMA,
        ],
    )
    def gather_kernel(table_hbm, idx_hbm, out_hbm, idx_v, buf0, buf1, gsem, osem0, osem1):
        wid = lax.axis_index("subcore") * NUM_CORES + lax.axis_index("core")
        row_base = wid * rows_per_worker

        pltpu.sync_copy(idx_hbm.at[0, pl.ds(row_base * seq, idx_per_worker)], idx_v)

        bufs = (buf0, buf1)
        osems = (osem0, osem1)

        def start_gather(g):
            return pltpu.async_copy(
                table_hbm.at[idx_v.at[pl.ds(g * gw, gw)]], bufs[g % 2], gsem
            )

        out_handles = [[], []]
        gather_handle = start_gather(0)
        for g in range(n_gathers):
            gather_handle.wait()
            if g + 1 < n_gathers:
                nxt = (g + 1) % 2
                for h in out_handles[nxt]:
                    h.wait()
                out_handles[nxt] = []
                gather_handle = start_gather(g + 1)
            buf = bufs[g % 2]
            for j in range(ROWS_PER_GATHER):
                out_handles[g % 2].append(
                    pltpu.async_copy(
                        buf.at[pl.ds(j * seq, seq)],
                        out_hbm.at[row_base + g * ROWS_PER_GATHER + j],
                        osems[g % 2],
                    )
                )
        for side in out_handles:
            for h in side:
                h.wait()

    return gather_kernel(table, idx)
